# speedup vs baseline: 1.6637x; 1.6637x over previous
import jax
import jax.numpy as jnp
from jax.experimental import pallas as pl

_B = 4
_TRACK = 8192
_D = 1024
_R = 1024


def _add_body(x_ref, t_ref, o_ref):
    o_ref[...] = x_ref[...] + t_ref[...]


def kernel(inputs, table):
    return pl.pallas_call(
        _add_body,
        grid=(_TRACK // _R, _B),
        in_specs=[
            pl.BlockSpec((1, _R, _D), lambda i, b: (b, i, 0)),
            pl.BlockSpec((_R, _D), lambda i, b: (i, 0)),
        ],
        out_specs=pl.BlockSpec((1, _R, _D), lambda i, b: (b, i, 0)),
        out_shape=jax.ShapeDtypeStruct(inputs.shape, inputs.dtype),
    )(inputs, table)
